# baseline (speedup 1.0000x reference)
import functools

import jax
import jax.numpy as jnp
from jax import lax
from jax.experimental import pallas as pl
from jax.experimental.pallas import tpu as pltpu
from jax.experimental.pallas import tpu_sc as plsc

_N_TOKENS = 10
_N_PROMPTS = 64
_EMBED = 64
_BATCH = 4096
_SEQ = 210

_NC = 2
_NS = 16
_NW = _NC * _NS
_L = 16

_TROWS = _N_PROMPTS + _N_PROMPTS * _N_TOKENS
_BC = 256
_NBC = _BATCH // _BC
_CHUNKS = _SEQ * _NBC
_CPW = _CHUNKS // _NW


def _sc_embed(cidx3, table_flat):
    mesh = plsc.VectorSubcoreMesh(core_axis_name="c", subcore_axis_name="s")

    @functools.partial(
        pl.kernel,
        mesh=mesh,
        compiler_params=pltpu.CompilerParams(needs_layout_passes=False),
        out_type=jax.ShapeDtypeStruct((_SEQ, _EMBED, _BATCH), jnp.float32),
        scratch_types=[
            pltpu.VMEM((_TROWS * _EMBED,), jnp.float32),
            pltpu.VMEM((1, _CPW, _BC), jnp.int32),
            pltpu.VMEM((2, 1, _EMBED, _BC), jnp.float32),
            pltpu.SemaphoreType.DMA,
        ],
    )
    def run(cidx_hbm, table_hbm, out_hbm, tab_v, idx_v, stage, ssem):
        wid = lax.axis_index("s") * _NC + lax.axis_index("c")
        pltpu.sync_copy(table_hbm, tab_v)
        pltpu.sync_copy(cidx_hbm.at[pl.ds(wid, 1)], idx_v)
        cid0 = wid * _CPW

        def chunk(j, carry):
            cid = cid0 + j
            p = lax.div(cid, _NBC)
            b0 = lax.rem(cid, _NBC) * _BC
            buf = lax.rem(j, 2)

            @pl.when(j >= 2)
            def _wait_prior():
                pltpu.make_async_copy(
                    stage.at[buf],
                    out_hbm.at[pl.ds(0, 1), slice(None), pl.ds(0, _BC)],
                    ssem,
                ).wait()

            def group(g, c2):
                bo = g * _L
                base = idx_v[0, j, pl.ds(bo, _L)]
                for e in range(_EMBED):
                    v = plsc.load_gather(tab_v, [base + e])
                    stage[buf, 0, e, pl.ds(bo, _L)] = v
                return c2

            lax.fori_loop(0, _BC // _L, group, 0)

            pltpu.make_async_copy(
                stage.at[buf],
                out_hbm.at[pl.ds(p, 1), slice(None), pl.ds(b0, _BC)],
                ssem,
            ).start()
            return carry

        lax.fori_loop(0, _CPW, chunk, 0)

        for _ in range(2):
            pltpu.make_async_copy(
                stage.at[0],
                out_hbm.at[pl.ds(0, 1), slice(None), pl.ds(0, _BC)],
                ssem,
            ).wait()

    return run(cidx3, table_flat)


@jax.jit
def kernel(tokens, wte_weight, learned_embedding):
    tokens = tokens.astype(jnp.int32)
    tokT = tokens.T
    pos = jnp.arange(_SEQ, dtype=jnp.int32)[:, None]
    cidxT = jnp.where(
        pos >= _N_TOKENS,
        tokT,
        _N_PROMPTS + tokT[0:1, :] * _N_TOKENS + pos,
    )
    cidx3 = (cidxT * _EMBED).reshape(_NW, _CPW, _BC)
    table_flat = jnp.concatenate(
        [wte_weight[:_N_PROMPTS], learned_embedding.reshape(-1, _EMBED)], axis=0
    ).reshape(-1)
    outT = _sc_embed(cidx3, table_flat)
    return jnp.transpose(outT, (2, 0, 1))


# speedup vs baseline: 1.5843x; 1.5843x over previous
import functools

import jax
import jax.numpy as jnp
from jax import lax
from jax.experimental import pallas as pl
from jax.experimental.pallas import tpu as pltpu
from jax.experimental.pallas import tpu_sc as plsc

_N_TOKENS = 10
_N_PROMPTS = 64
_EMBED = 64
_BATCH = 4096
_SEQ = 210

_NC = 2
_NS = 16
_NW = _NC * _NS
_L = 16

_TROWS = _N_PROMPTS + _N_PROMPTS * _N_TOKENS
_BC = 256
_NBC = _BATCH // _BC
_CHUNKS = _SEQ * _NBC
_CPW = _CHUNKS // _NW


def _sc_embed(cidx3, table_flat):
    mesh = plsc.VectorSubcoreMesh(core_axis_name="c", subcore_axis_name="s")

    @functools.partial(
        pl.kernel,
        mesh=mesh,
        compiler_params=pltpu.CompilerParams(needs_layout_passes=False),
        out_type=jax.ShapeDtypeStruct((_SEQ, _EMBED, _BATCH), jnp.float32),
        scratch_types=[
            pltpu.VMEM((_TROWS * _EMBED,), jnp.float32),
            pltpu.VMEM((1, _CPW, _BC), jnp.int32),
            pltpu.VMEM((2, 1, _EMBED, _BC), jnp.float32),
            pltpu.SemaphoreType.DMA,
        ],
    )
    def run(cidx_hbm, table_hbm, out_hbm, tab_v, idx_v, stage, ssem):
        wid = lax.axis_index("s") * _NC + lax.axis_index("c")
        pltpu.sync_copy(table_hbm, tab_v)
        pltpu.sync_copy(cidx_hbm.at[pl.ds(wid, 1)], idx_v)
        cid0 = wid * _CPW

        def chunk(j, carry):
            cid = cid0 + j
            p = lax.div(cid, _NBC)
            b0 = lax.rem(cid, _NBC) * _BC
            buf = lax.rem(j, 2)

            @pl.when(j >= 2)
            def _wait_prior():
                pltpu.make_async_copy(
                    stage.at[buf],
                    out_hbm.at[pl.ds(0, 1), slice(None), pl.ds(0, _BC)],
                    ssem,
                ).wait()

            @plsc.parallel_loop(0, _BC // _L, 1, unroll=2)
            def _group(g):
                bo = g * _L
                base = idx_v[0, j, pl.ds(bo, _L)]
                for e in range(_EMBED):
                    v = plsc.load_gather(tab_v, [base + e])
                    stage[buf, 0, e, pl.ds(bo, _L)] = v

            pltpu.make_async_copy(
                stage.at[buf],
                out_hbm.at[pl.ds(p, 1), slice(None), pl.ds(b0, _BC)],
                ssem,
            ).start()
            return carry

        lax.fori_loop(0, _CPW, chunk, 0)

        for _ in range(2):
            pltpu.make_async_copy(
                stage.at[0],
                out_hbm.at[pl.ds(0, 1), slice(None), pl.ds(0, _BC)],
                ssem,
            ).wait()

    return run(cidx3, table_flat)


@jax.jit
def kernel(tokens, wte_weight, learned_embedding):
    tokens = tokens.astype(jnp.int32)
    tokT = tokens.T
    pos = jnp.arange(_SEQ, dtype=jnp.int32)[:, None]
    cidxT = jnp.where(
        pos >= _N_TOKENS,
        tokT,
        _N_PROMPTS + tokT[0:1, :] * _N_TOKENS + pos,
    )
    cidx3 = (cidxT * _EMBED).reshape(_NW, _CPW, _BC)
    table_flat = jnp.concatenate(
        [wte_weight[:_N_PROMPTS], learned_embedding.reshape(-1, _EMBED)], axis=0
    ).reshape(-1)
    outT = _sc_embed(cidx3, table_flat)
    return jnp.transpose(outT, (2, 0, 1))


# speedup vs baseline: 5.8927x; 3.7195x over previous
import functools

import jax
import jax.numpy as jnp
from jax import lax
from jax.experimental import pallas as pl
from jax.experimental.pallas import tpu as pltpu
from jax.experimental.pallas import tpu_sc as plsc

_N_TOKENS = 10
_N_PROMPTS = 64
_EMBED = 64
_BATCH = 4096
_SEQ = 210

_NC = 2
_NS = 16
_NW = _NC * _NS
_L = 16

_TROWS = _N_PROMPTS + _N_PROMPTS * _N_TOKENS
_TSTRIDE = _EMBED + 1
_BC = 256
_NBC = _BATCH // _BC
_CHUNKS = _SEQ * _NBC
_CPW = _CHUNKS // _NW


def _sc_embed(cidx3, table_flat):
    mesh = plsc.VectorSubcoreMesh(core_axis_name="c", subcore_axis_name="s")

    @functools.partial(
        pl.kernel,
        mesh=mesh,
        compiler_params=pltpu.CompilerParams(needs_layout_passes=False),
        out_type=jax.ShapeDtypeStruct((_SEQ, _EMBED, _BATCH), jnp.float32),
        scratch_types=[
            pltpu.VMEM((_TROWS * _TSTRIDE,), jnp.float32),
            pltpu.VMEM((1, _CPW, _BC), jnp.int32),
            pltpu.VMEM((2, 1, _EMBED, _BC), jnp.float32),
            pltpu.SemaphoreType.DMA,
        ],
    )
    def run(cidx_hbm, table_hbm, out_hbm, tab_v, idx_v, stage, ssem):
        wid = lax.axis_index("s") * _NC + lax.axis_index("c")
        pltpu.sync_copy(table_hbm, tab_v)
        pltpu.sync_copy(cidx_hbm.at[pl.ds(wid, 1)], idx_v)
        cid0 = wid * _CPW

        def chunk(j, carry):
            cid = cid0 + j
            p = lax.div(cid, _NBC)
            b0 = lax.rem(cid, _NBC) * _BC
            buf = lax.rem(j, 2)

            @pl.when(j >= 2)
            def _wait_prior():
                pltpu.make_async_copy(
                    stage.at[buf],
                    out_hbm.at[pl.ds(0, 1), slice(None), pl.ds(0, _BC)],
                    ssem,
                ).wait()

            @plsc.parallel_loop(0, _BC // _L, 1, unroll=2)
            def _group(g):
                bo = g * _L
                base = idx_v[0, j, pl.ds(bo, _L)]
                for e in range(_EMBED):
                    v = plsc.load_gather(tab_v, [base + e])
                    stage[buf, 0, e, pl.ds(bo, _L)] = v

            pltpu.make_async_copy(
                stage.at[buf],
                out_hbm.at[pl.ds(p, 1), slice(None), pl.ds(b0, _BC)],
                ssem,
            ).start()
            return carry

        lax.fori_loop(0, _CPW, chunk, 0)

        for _ in range(2):
            pltpu.make_async_copy(
                stage.at[0],
                out_hbm.at[pl.ds(0, 1), slice(None), pl.ds(0, _BC)],
                ssem,
            ).wait()

    return run(cidx3, table_flat)


@jax.jit
def kernel(tokens, wte_weight, learned_embedding):
    tokens = tokens.astype(jnp.int32)
    tokT = tokens.T
    pos = jnp.arange(_SEQ, dtype=jnp.int32)[:, None]
    cidxT = jnp.where(
        pos >= _N_TOKENS,
        tokT,
        _N_PROMPTS + tokT[0:1, :] * _N_TOKENS + pos,
    )
    cidx3 = (cidxT * _TSTRIDE).reshape(_NW, _CPW, _BC)
    table_flat = jnp.pad(
        jnp.concatenate(
            [wte_weight[:_N_PROMPTS], learned_embedding.reshape(-1, _EMBED)],
            axis=0,
        ),
        ((0, 0), (0, _TSTRIDE - _EMBED)),
    ).reshape(-1)
    outT = _sc_embed(cidx3, table_flat)
    return jnp.transpose(outT, (2, 0, 1))
